# baseline (speedup 1.0000x reference)
import functools

import jax
import jax.numpy as jnp
from jax import lax
from jax.experimental import pallas as pl
from jax.experimental.pallas import tpu as pltpu
from jax.experimental.pallas import tpu_sc as plsc

B = 16384
D = 32
L = 16
NC = 2
NS = 16
NW = NC * NS
BPW = B // NW
NG = BPW // L
NT = 125000

_mesh = plsc.VectorSubcoreMesh(core_axis_name="c", subcore_axis_name="s")


@functools.partial(
    pl.kernel,
    mesh=_mesh,
    compiler_params=pltpu.CompilerParams(needs_layout_passes=False),
    out_type=jax.ShapeDtypeStruct((B,), jnp.float32),
    scratch_types=[
        pltpu.VMEM((BPW,), jnp.int32),
        pltpu.VMEM((BPW,), jnp.int32),
        pltpu.VMEM_SHARED((16, 2, L, D), jnp.float32),
        pltpu.VMEM_SHARED((16, 2, L, D), jnp.float32),
        pltpu.VMEM((L, D), jnp.float32),
        pltpu.VMEM((L, D), jnp.float32),
        pltpu.VMEM((BPW,), jnp.float32),
        pltpu.VMEM((L * (L + 1),), jnp.float32),
        pltpu.SemaphoreType.DMA,
        pltpu.SemaphoreType.DMA,
    ],
)
def _sc_forward(users_hbm, items_hbm, ut_hbm, it_hbm, out_hbm,
                uidx_v, iidx_v, ush_v, ish_v, ubuf_v, ibuf_v,
                out_v, t_v, sem_u, sem_i):
    sid = lax.axis_index("s")
    wid = sid * NC + lax.axis_index("c")
    base = wid * BPW

    pltpu.sync_copy(users_hbm.at[pl.ds(base, BPW)], uidx_v)
    pltpu.sync_copy(items_hbm.at[pl.ds(base, BPW)], iidx_v)

    lanes17 = lax.iota(jnp.int32, L) * (L + 1)

    def fire(g, slot):
        uvg = uidx_v[pl.ds(g * L, L)]
        ivg = iidx_v[pl.ds(g * L, L)]
        ut = jax.lax.shift_right_logical(uvg, 3)
        it = jax.lax.shift_right_logical(ivg, 3)
        us = uvg & 7
        ws = ivg & 7
        for j in range(L):
            pltpu.async_copy(
                ut_hbm.at[ut[j], us[j], :], ush_v.at[sid, slot, j], sem_u)
            pltpu.async_copy(
                it_hbm.at[it[j], ws[j], :], ish_v.at[sid, slot, j], sem_i)

    def drain(slot):
        for j in range(L):
            pltpu.make_async_copy(
                ut_hbm.at[0, 0, :], ush_v.at[sid, slot, j], sem_u).wait()
            pltpu.make_async_copy(
                it_hbm.at[0, 0, :], ish_v.at[sid, slot, j], sem_i).wait()

    def compute(g, slot):
        row0 = g * L
        pltpu.sync_copy(ush_v.at[sid, slot], ubuf_v)
        pltpu.sync_copy(ish_v.at[sid, slot], ibuf_v)
        for j in range(L):
            lo = (ubuf_v[j, pl.ds(0, L)]
                  * ibuf_v[j, pl.ds(0, L)])
            hi = (ubuf_v[j, pl.ds(L, L)]
                  * ibuf_v[j, pl.ds(L, L)])
            plsc.store_scatter(t_v, [lanes17 + j], lo + hi)
        acc = t_v[pl.ds(0, L)]
        for l in range(1, L):
            acc = acc + t_v[pl.ds(l * (L + 1), L)]
        out_v[pl.ds(row0, L)] = acc

    fire(0, 0)

    def pair_body(h, carry):
        g0 = 2 * h
        fire(g0 + 1, 1)
        drain(0)
        compute(g0, 0)
        fire(lax.rem(g0 + 2, NG), 0)
        drain(1)
        compute(g0 + 1, 1)
        return carry

    lax.fori_loop(0, NG // 2, pair_body, 0)
    drain(0)

    pltpu.sync_copy(out_v, out_hbm.at[pl.ds(base, BPW)])


def kernel(users, items, user_table, item_table):
    ut3 = user_table.reshape(NT, 8, D)
    it3 = item_table.reshape(NT, 8, D)
    return _sc_forward(users, items, ut3, it3)


# speedup vs baseline: 1.1484x; 1.1484x over previous
import functools

import jax
import jax.numpy as jnp
from jax import lax
from jax.experimental import pallas as pl
from jax.experimental.pallas import tpu as pltpu
from jax.experimental.pallas import tpu_sc as plsc

B = 16384
D = 32
L = 16
NC = 2
NS = 16
NW = NC * NS
BPW = B // NW
NG = BPW // L
NT = 125000

_mesh = plsc.VectorSubcoreMesh(core_axis_name="c", subcore_axis_name="s")


@functools.partial(
    pl.kernel,
    mesh=_mesh,
    compiler_params=pltpu.CompilerParams(needs_layout_passes=False),
    out_type=jax.ShapeDtypeStruct((B,), jnp.float32),
    scratch_types=[
        pltpu.VMEM((BPW,), jnp.int32),
        pltpu.VMEM((BPW,), jnp.int32),
        pltpu.VMEM((2, L, D), jnp.float32),
        pltpu.VMEM((2, L, D), jnp.float32),
        pltpu.VMEM((BPW,), jnp.float32),
        pltpu.VMEM((L * (L + 1),), jnp.float32),
        pltpu.SemaphoreType.DMA,
        pltpu.SemaphoreType.DMA,
    ],
)
def _sc_forward(users_hbm, items_hbm, ut_hbm, it_hbm, out_hbm,
                uidx_v, iidx_v, ubuf_v, ibuf_v, out_v, t_v, sem_u, sem_i):
    wid = lax.axis_index("s") * NC + lax.axis_index("c")
    base = wid * BPW

    pltpu.sync_copy(users_hbm.at[pl.ds(base, BPW)], uidx_v)
    pltpu.sync_copy(items_hbm.at[pl.ds(base, BPW)], iidx_v)

    lanes17 = lax.iota(jnp.int32, L) * (L + 1)

    def fire(g, slot):
        uvg = uidx_v[pl.ds(g * L, L)]
        ivg = iidx_v[pl.ds(g * L, L)]
        ut = jax.lax.shift_right_logical(uvg, 3)
        it = jax.lax.shift_right_logical(ivg, 3)
        us = uvg & 7
        ws = ivg & 7
        for j in range(L):
            pltpu.async_copy(
                ut_hbm.at[ut[j], us[j], :], ubuf_v.at[slot, j], sem_u)
            pltpu.async_copy(
                it_hbm.at[it[j], ws[j], :], ibuf_v.at[slot, j], sem_i)

    def drain(slot):
        for j in range(L):
            pltpu.make_async_copy(
                ut_hbm.at[0, 0, :], ubuf_v.at[slot, j], sem_u).wait()
            pltpu.make_async_copy(
                it_hbm.at[0, 0, :], ibuf_v.at[slot, j], sem_i).wait()

    def compute(g, slot):
        row0 = g * L
        for j in range(L):
            lo = (ubuf_v[slot, j, pl.ds(0, L)]
                  * ibuf_v[slot, j, pl.ds(0, L)])
            hi = (ubuf_v[slot, j, pl.ds(L, L)]
                  * ibuf_v[slot, j, pl.ds(L, L)])
            plsc.store_scatter(t_v, [lanes17 + j], lo + hi)
        acc = t_v[pl.ds(0, L)]
        for l in range(1, L):
            acc = acc + t_v[pl.ds(l * (L + 1), L)]
        out_v[pl.ds(row0, L)] = acc

    fire(0, 0)

    def pair_body(h, carry):
        g0 = 2 * h
        fire(g0 + 1, 1)
        drain(0)
        compute(g0, 0)
        fire(lax.rem(g0 + 2, NG), 0)
        drain(1)
        compute(g0 + 1, 1)
        return carry

    lax.fori_loop(0, NG // 2, pair_body, 0)
    drain(0)

    pltpu.sync_copy(out_v, out_hbm.at[pl.ds(base, BPW)])


def kernel(users, items, user_table, item_table):
    ut3 = user_table.reshape(NT, 8, D)
    it3 = item_table.reshape(NT, 8, D)
    return _sc_forward(users, items, ut3, it3)
